# Initial kernel scaffold; baseline (speedup 1.0000x reference)
import jax
import jax.numpy as jnp
from jax.experimental import pallas as pl


def kernel(x, mapping, luts):
    raise NotImplementedError("write your pallas kernel here")


# speedup vs baseline: 92.1999x; 92.1999x over previous
import functools
import jax
import jax.numpy as jnp
from jax import lax
from jax.experimental import pallas as pl
from jax.experimental.pallas import tpu as pltpu
from jax.experimental.pallas import tpu_sc as plsc

_INPUT = 2048
_OUT = 1024
_NBITS = 6
_BATCH = 512
_NLUT = 1 << _NBITS
_NW = 32
_BPW = _BATCH // _NW
_L = 16
_NCHUNK = _OUT // _L


def _lut_body(x_hbm, mapt_hbm, luts_hbm, out_hbm, x_v, mapt_v, luts_v, out_v):
    wid = lax.axis_index("s") * 2 + lax.axis_index("c")
    base = wid * _BPW
    pltpu.sync_copy(x_hbm.at[pl.ds(base * _INPUT, _BPW * _INPUT)], x_v)
    pltpu.sync_copy(mapt_hbm, mapt_v)
    pltpu.sync_copy(luts_hbm, luts_v)

    jiota = lax.iota(jnp.int32, _L)

    def jc_body(jc, carry):
        jb = jc * _L
        lbase = (jiota + jb) * _NLUT
        idxs = [mapt_v[pl.ds(k * _OUT + jb, _L)] for k in range(_NBITS)]
        for b in range(_BPW):
            addr = jnp.zeros((_L,), jnp.int32)
            for k in range(_NBITS):
                g = plsc.load_gather(x_v, [idxs[k] + (b * _INPUT)])
                addr = addr + jnp.where(g > 0.0, jnp.int32(1 << k), jnp.int32(0))
            lv = plsc.load_gather(luts_v, [lbase + addr])
            out_v[pl.ds(b * _OUT + jb, _L)] = jnp.where(lv > 0.0, 1.0, 0.0)
        return carry

    lax.fori_loop(0, _NCHUNK, jc_body, 0)
    pltpu.sync_copy(out_v, out_hbm.at[pl.ds(base * _OUT, _BPW * _OUT)])


@jax.jit
def _lut_forward(x, mapping_t, luts):
    mesh = plsc.VectorSubcoreMesh(core_axis_name="c", subcore_axis_name="s")
    fn = functools.partial(
        pl.kernel,
        mesh=mesh,
        compiler_params=pltpu.CompilerParams(needs_layout_passes=False),
        out_type=jax.ShapeDtypeStruct((_BATCH * _OUT,), jnp.float32),
        scratch_types=[
            pltpu.VMEM((_BPW * _INPUT,), jnp.float32),
            pltpu.VMEM((_NBITS * _OUT,), jnp.int32),
            pltpu.VMEM((_OUT * _NLUT,), jnp.float32),
            pltpu.VMEM((_BPW * _OUT,), jnp.float32),
        ],
    )(_lut_body)
    return fn(x.reshape(-1), mapping_t.reshape(-1), luts.reshape(-1))


def kernel(x, mapping, luts):
    out = _lut_forward(x, mapping.T, luts)
    return out.reshape(_BATCH, _OUT)


# speedup vs baseline: 113.9580x; 1.2360x over previous
import functools
import jax
import jax.numpy as jnp
from jax import lax
from jax.experimental import pallas as pl
from jax.experimental.pallas import tpu as pltpu
from jax.experimental.pallas import tpu_sc as plsc

_INPUT = 2048
_OUT = 1024
_NBITS = 6
_BATCH = 512
_NLUT = 1 << _NBITS
_NW = 32
_BPW = _BATCH // _NW
_L = 16
_NCHUNK = _OUT // _L


def _lut_body(x_hbm, mapt_hbm, luts_hbm, out_hbm, x_v, mapt_v, luts_v, out_v):
    wid = lax.axis_index("s") * 2 + lax.axis_index("c")
    base = wid * _BPW
    pltpu.sync_copy(x_hbm.at[pl.ds(base * _INPUT, _BPW * _INPUT)], x_v)
    pltpu.sync_copy(mapt_hbm, mapt_v)
    pltpu.sync_copy(luts_hbm, luts_v)

    jiota = lax.iota(jnp.int32, _L)

    @plsc.parallel_loop(0, _NCHUNK)
    def jc_body(jc):
        jb = jc * _L
        lbase = (jiota + jb) * _NLUT
        idxs = [mapt_v[pl.ds(k * _OUT + jb, _L)] for k in range(_NBITS)]
        for b in range(_BPW):
            addr = jnp.zeros((_L,), jnp.int32)
            for k in range(_NBITS):
                g = plsc.load_gather(x_v, [idxs[k] + (b * _INPUT)])
                addr = addr + jnp.where(g > 0.0, jnp.int32(1 << k), jnp.int32(0))
            lv = plsc.load_gather(luts_v, [lbase + addr])
            out_v[pl.ds(b * _OUT + jb, _L)] = jnp.where(lv > 0.0, 1.0, 0.0)
    pltpu.sync_copy(out_v, out_hbm.at[pl.ds(base * _OUT, _BPW * _OUT)])


@jax.jit
def _lut_forward(x, mapping_t, luts):
    mesh = plsc.VectorSubcoreMesh(core_axis_name="c", subcore_axis_name="s")
    fn = functools.partial(
        pl.kernel,
        mesh=mesh,
        compiler_params=pltpu.CompilerParams(needs_layout_passes=False),
        out_type=jax.ShapeDtypeStruct((_BATCH * _OUT,), jnp.float32),
        scratch_types=[
            pltpu.VMEM((_BPW * _INPUT,), jnp.float32),
            pltpu.VMEM((_NBITS * _OUT,), jnp.int32),
            pltpu.VMEM((_OUT * _NLUT,), jnp.float32),
            pltpu.VMEM((_BPW * _OUT,), jnp.float32),
        ],
    )(_lut_body)
    return fn(x.reshape(-1), mapping_t.reshape(-1), luts.reshape(-1))


def kernel(x, mapping, luts):
    out = _lut_forward(x, mapping.T, luts)
    return out.reshape(_BATCH, _OUT)
